# Initial kernel scaffold; baseline (speedup 1.0000x reference)
import functools

import jax
import jax.numpy as jnp
from jax.experimental import pallas as pl
from jax.experimental.pallas import tpu as pltpu

_IMG = 512.0
_N = 20000
_CLS = 80
_PRE_K = 1000
_POST_K = 100
_PAD_K = 1024
_IOU_THR = 0.5

_f32 = jnp.float32


def _score_body(scores_ref, sc_ref):
    sc_ref[...] = jnp.max(jax.nn.sigmoid(scores_ref[...]), axis=1)


def _dot(a, b, dims):
    return jax.lax.dot_general(
        a, b, (dims, ((), ())),
        precision=jax.lax.Precision.HIGHEST,
        preferred_element_type=_f32,
    )


def _decode(x1, y1, x2, y2, d0, d1, d2, d3):
    px = (x1 + x2) * 0.5
    py = (y1 + y2) * 0.5
    pw = x2 - x1
    ph = y2 - y1
    dw = jnp.clip(d2, -4.0, 4.0)
    dh = jnp.clip(d3, -4.0, 4.0)
    gx = px + pw * d0
    gy = py + ph * d1
    gw = pw * jnp.exp(dw)
    gh = ph * jnp.exp(dh)
    cx1 = jnp.clip(gx - gw * 0.5, 0.0, _IMG)
    cy1 = jnp.clip(gy - gh * 0.5, 0.0, _IMG)
    cx2 = jnp.clip(gx + gw * 0.5, 0.0, _IMG)
    cy2 = jnp.clip(gy + gh * 0.5, 0.0, _IMG)
    return cx1, cy1, cx2, cy2


def _nms_body(sc_col_ref, cand_ref, candt_ref, cdel_ref, cdelt_ref,
              out_ref, iou_ref):
    cx1, cy1, cx2, cy2 = _decode(
        cand_ref[:, 0:1], cand_ref[:, 1:2], cand_ref[:, 2:3], cand_ref[:, 3:4],
        cdel_ref[:, 0:1], cdel_ref[:, 1:2], cdel_ref[:, 2:3], cdel_ref[:, 3:4])
    rx1, ry1, rx2, ry2 = _decode(
        candt_ref[0:1, :], candt_ref[1:2, :], candt_ref[2:3, :], candt_ref[3:4, :],
        cdelt_ref[0:1, :], cdelt_ref[1:2, :], cdelt_ref[2:3, :], cdelt_ref[3:4, :])

    ltx = jnp.maximum(cx1, rx1)
    lty = jnp.maximum(cy1, ry1)
    rbx = jnp.minimum(cx2, rx2)
    rby = jnp.minimum(cy2, ry2)
    wx = jnp.clip(rbx - ltx, 0.0, None)
    wy = jnp.clip(rby - lty, 0.0, None)
    inter = wx * wy
    a_col = (cx2 - cx1) * (cy2 - cy1)
    a_row = (rx2 - rx1) * (ry2 - ry1)
    iou_ref[...] = inter / (((a_col + a_row) - inter) + 1e-6)

    lidx = jax.lax.broadcasted_iota(jnp.int32, (1, _PAD_K), 1)

    def body(i, supp):
        row = iou_ref[pl.ds(i, 1), :]
        s_i = jnp.sum(jnp.where(lidx == i, supp, 0.0))
        mask = (row > _IOU_THR) & (lidx > i)
        new = jnp.where(mask, 1.0, supp)
        return jnp.where(s_i > 0.0, supp, new)

    supp = jax.lax.fori_loop(0, _PRE_K, body, jnp.zeros((1, _PAD_K), _f32),
                             unroll=4)

    padm = lidx >= _PRE_K
    u_row = jnp.where((supp == 0.0) & ~padm, 1.0, 0.0)
    s_row = jnp.where((supp > 0.0) & ~padm, 1.0, 0.0)
    i0 = jax.lax.broadcasted_iota(jnp.int32, (_PAD_K, _PAD_K), 0)
    i1 = jax.lax.broadcasted_iota(jnp.int32, (_PAD_K, _PAD_K), 1)
    lt_mat = jnp.where(i0 < i1, 1.0, 0.0)
    eye = jnp.where(i0 == i1, 1.0, 0.0)
    ucum = _dot(u_row, lt_mat, ((1,), (0,)))
    scum = _dot(s_row, lt_mat, ((1,), (0,)))
    num_u = jnp.sum(u_row)
    rank = jnp.where(padm, 5000.0, jnp.where(supp > 0.0, num_u + scum, ucum))

    r_iota = jax.lax.broadcasted_iota(_f32, (_POST_K, _PAD_K), 0)
    oneh = jnp.where(r_iota == rank, 1.0, 0.0)

    supp_col = _dot(eye, supp, ((1,), (1,)))
    ridx = jax.lax.broadcasted_iota(jnp.int32, (_PAD_K, 1), 0)
    k_col = jnp.where(ridx >= _PRE_K, -2.0,
                      jnp.where(supp_col > 0.0, -1.0, sc_col_ref[...]))
    feat = jnp.concatenate([cx1, cy1, cx2, cy2, k_col], axis=1)
    out_ref[...] = _dot(oneh, feat, ((1,), (0,)))


def kernel(boxes, deltas, scores):
    sc = pl.pallas_call(
        _score_body,
        out_shape=jax.ShapeDtypeStruct((_N,), _f32),
    )(scores)

    top_sc, top_idx = jax.lax.top_k(sc, _PRE_K)
    cand = jnp.pad(boxes[top_idx], ((0, _PAD_K - _PRE_K), (0, 0)))
    cdel = jnp.pad(deltas[top_idx], ((0, _PAD_K - _PRE_K), (0, 0)))
    sc_col = jnp.pad(top_sc, (0, _PAD_K - _PRE_K),
                     constant_values=-2.0).reshape(_PAD_K, 1)

    return pl.pallas_call(
        _nms_body,
        out_shape=jax.ShapeDtypeStruct((_POST_K, 5), _f32),
        scratch_shapes=[pltpu.VMEM((_PAD_K, _PAD_K), _f32)],
    )(sc_col, cand, cand.T, cdel, cdel.T)


# speedup vs baseline: 7.3673x; 7.3673x over previous
import functools

import jax
import jax.numpy as jnp
from jax.experimental import pallas as pl
from jax.experimental.pallas import tpu as pltpu

_IMG = 512.0
_N = 20000
_CLS = 80
_PRE_K = 1000
_POST_K = 100
_PAD_K = 1024
_IOU_THR = 0.5

_f32 = jnp.float32


def _score_body(scores_ref, sc_ref):
    sc_ref[...] = jnp.max(jax.nn.sigmoid(scores_ref[...]), axis=1)


def _dot(a, b, dims):
    return jax.lax.dot_general(
        a, b, (dims, ((), ())),
        precision=jax.lax.Precision.HIGHEST,
        preferred_element_type=_f32,
    )


def _decode(x1, y1, x2, y2, d0, d1, d2, d3):
    px = (x1 + x2) * 0.5
    py = (y1 + y2) * 0.5
    pw = x2 - x1
    ph = y2 - y1
    dw = jnp.clip(d2, -4.0, 4.0)
    dh = jnp.clip(d3, -4.0, 4.0)
    gx = px + pw * d0
    gy = py + ph * d1
    gw = pw * jnp.exp(dw)
    gh = ph * jnp.exp(dh)
    cx1 = jnp.clip(gx - gw * 0.5, 0.0, _IMG)
    cy1 = jnp.clip(gy - gh * 0.5, 0.0, _IMG)
    cx2 = jnp.clip(gx + gw * 0.5, 0.0, _IMG)
    cy2 = jnp.clip(gy + gh * 0.5, 0.0, _IMG)
    return cx1, cy1, cx2, cy2


def _nms_body(sc_col_ref, cand_ref, candt_ref, cdel_ref, cdelt_ref,
              out_ref, iou_ref):
    cx1, cy1, cx2, cy2 = _decode(
        cand_ref[:, 0:1], cand_ref[:, 1:2], cand_ref[:, 2:3], cand_ref[:, 3:4],
        cdel_ref[:, 0:1], cdel_ref[:, 1:2], cdel_ref[:, 2:3], cdel_ref[:, 3:4])
    rx1, ry1, rx2, ry2 = _decode(
        candt_ref[0:1, :], candt_ref[1:2, :], candt_ref[2:3, :], candt_ref[3:4, :],
        cdelt_ref[0:1, :], cdelt_ref[1:2, :], cdelt_ref[2:3, :], cdelt_ref[3:4, :])

    ltx = jnp.maximum(cx1, rx1)
    lty = jnp.maximum(cy1, ry1)
    rbx = jnp.minimum(cx2, rx2)
    rby = jnp.minimum(cy2, ry2)
    wx = jnp.clip(rbx - ltx, 0.0, None)
    wy = jnp.clip(rby - lty, 0.0, None)
    inter = wx * wy
    a_col = (cx2 - cx1) * (cy2 - cy1)
    a_row = (rx2 - rx1) * (ry2 - ry1)
    iou_ref[...] = inter / (((a_col + a_row) - inter) + 1e-6)

    lidx = jax.lax.broadcasted_iota(jnp.int32, (1, _PAD_K), 1)

    def body(i, supp):
        row = iou_ref[pl.ds(i, 1), :]
        s_i = jnp.sum(jnp.where(lidx == i, supp, 0.0))
        mask = (row > _IOU_THR) & (lidx > i)
        new = jnp.where(mask, 1.0, supp)
        return jnp.where(s_i > 0.0, supp, new)

    supp = jax.lax.fori_loop(0, _PRE_K, body, jnp.zeros((1, _PAD_K), _f32),
                             unroll=4)

    padm = lidx >= _PRE_K
    u_row = jnp.where((supp == 0.0) & ~padm, 1.0, 0.0)
    s_row = jnp.where((supp > 0.0) & ~padm, 1.0, 0.0)
    i0 = jax.lax.broadcasted_iota(jnp.int32, (_PAD_K, _PAD_K), 0)
    i1 = jax.lax.broadcasted_iota(jnp.int32, (_PAD_K, _PAD_K), 1)
    lt_mat = jnp.where(i0 < i1, 1.0, 0.0)
    eye = jnp.where(i0 == i1, 1.0, 0.0)
    ucum = _dot(u_row, lt_mat, ((1,), (0,)))
    scum = _dot(s_row, lt_mat, ((1,), (0,)))
    num_u = jnp.sum(u_row)
    rank = jnp.where(padm, 5000.0, jnp.where(supp > 0.0, num_u + scum, ucum))

    r_iota = jax.lax.broadcasted_iota(jnp.int32, (_POST_K, _PAD_K), 0).astype(_f32)
    oneh = jnp.where(r_iota == rank, 1.0, 0.0)

    supp_col = _dot(eye, supp, ((1,), (1,)))
    ridx = jax.lax.broadcasted_iota(jnp.int32, (_PAD_K, 1), 0)
    k_col = jnp.where(ridx >= _PRE_K, -2.0,
                      jnp.where(supp_col > 0.0, -1.0, sc_col_ref[...]))
    feat = jnp.concatenate([cx1, cy1, cx2, cy2, k_col], axis=1)
    out_ref[...] = _dot(oneh, feat, ((1,), (0,)))


def kernel(boxes, deltas, scores):
    sc = pl.pallas_call(
        _score_body,
        out_shape=jax.ShapeDtypeStruct((_N,), _f32),
    )(scores)

    top_sc, top_idx = jax.lax.top_k(sc, _PRE_K)
    cand = jnp.pad(boxes[top_idx], ((0, _PAD_K - _PRE_K), (0, 0)))
    cdel = jnp.pad(deltas[top_idx], ((0, _PAD_K - _PRE_K), (0, 0)))
    sc_col = jnp.pad(top_sc, (0, _PAD_K - _PRE_K),
                     constant_values=-2.0).reshape(_PAD_K, 1)

    return pl.pallas_call(
        _nms_body,
        out_shape=jax.ShapeDtypeStruct((_POST_K, 5), _f32),
        scratch_shapes=[pltpu.VMEM((_PAD_K, _PAD_K), _f32)],
    )(sc_col, cand, cand.T, cdel, cdel.T)
